# baseline (speedup 1.0000x reference)
import functools

import jax
import jax.numpy as jnp
from jax import lax
from jax.experimental import pallas as pl
from jax.experimental.pallas import tpu as pltpu
from jax.experimental.pallas import tpu_sc as plsc


def _softmax_body(x_ref, o_ref):
    x = x_ref[...]
    m = jnp.max(x, axis=1, keepdims=True)
    e = jnp.exp(x - m)
    o_ref[...] = e / jnp.sum(e, axis=1, keepdims=True)


def _softmax_tc(x):
    return pl.pallas_call(
        _softmax_body,
        out_shape=jax.ShapeDtypeStruct(x.shape, x.dtype),
    )(x)


def _make_gather_sc(B, S, C, n_workers):
    rows_per_w = B // n_workers
    mesh = plsc.VectorSubcoreMesh(core_axis_name="c", subcore_axis_name="s")
    nc = 2
    assert rows_per_w % 2 == 0 and rows_per_w >= 4

    @functools.partial(
        pl.kernel,
        mesh=mesh,
        compiler_params=pltpu.CompilerParams(use_tc_tiling_on_sc=False),
        out_type=jax.ShapeDtypeStruct((B, S, C), jnp.float32),
        scratch_types=[
            pltpu.VMEM((rows_per_w, S), jnp.int32),
            pltpu.VMEM((S, C), jnp.float32),
            pltpu.VMEM((S, C), jnp.float32),
            pltpu.SemaphoreType.DMA,
            pltpu.SemaphoreType.DMA,
        ],
    )
    def gather_kernel(probs_hbm, idx_hbm, out_hbm, idx_v, buf0, buf1, sem0, sem1):
        wid = lax.axis_index("s") * nc + lax.axis_index("c")
        base = wid * rows_per_w
        bufs = (buf0, buf1)
        sems = (sem0, sem1)
        pltpu.sync_copy(idx_hbm.at[pl.ds(base, rows_per_w)], idx_v)

        def step(bb, b, first):
            if not first:
                pltpu.make_async_copy(bufs[b], out_hbm.at[bb - 2], sems[b]).wait()
            pltpu.async_copy(probs_hbm.at[idx_v.at[bb - base]], bufs[b], sems[b]).wait()
            pltpu.async_copy(bufs[b], out_hbm.at[bb], sems[b])

        step(base, 0, True)
        step(base + 1, 1, True)

        @pl.loop(base + 2, base + rows_per_w, step=2)
        def _(bb):
            step(bb, 0, False)
            step(bb + 1, 1, False)

        for b, off in ((0, rows_per_w - 2), (1, rows_per_w - 1)):
            pltpu.make_async_copy(bufs[b], out_hbm.at[base + off], sems[b]).wait()

    return gather_kernel


def kernel(inputs_hour, catid_time_matrix):
    B, S = inputs_hour.shape
    T, C = catid_time_matrix.shape
    n_workers = 32

    probs = _softmax_tc(catid_time_matrix)
    idx = inputs_hour.astype(jnp.int32)
    gather = _make_gather_sc(B, S, C, n_workers)
    return gather(probs, idx)
